# baseline (speedup 1.0000x reference)
import functools

import jax
import jax.numpy as jnp
from jax.experimental import pallas as pl

N = 10000
E = 160000
C = 128
Z = 4
S = 9
J = 4
R = 8
AVG = 16.0

BE = 2000
BN = 2000


def _edge_coeff_body(elen_ref, sph_ref, W0_ref, b0_ref, W1_ref, b1_ref,
                     W2_ref, b2_ref, W3_ref, b3_ref, coeff_ref):
    x = elen_ref[...]
    x = jax.nn.silu(jnp.dot(x, W0_ref[...], preferred_element_type=jnp.float32) + b0_ref[...])
    x = x + jax.nn.silu(jnp.dot(x, W1_ref[...], preferred_element_type=jnp.float32) + b1_ref[...])
    x = x + jax.nn.silu(jnp.dot(x, W2_ref[...], preferred_element_type=jnp.float32) + b2_ref[...])
    rl = jnp.dot(x, W3_ref[...], preferred_element_type=jnp.float32) + b3_ref[...]
    sph = sph_ref[...]
    acc = jnp.zeros((BE, J), jnp.float32)
    for s in range(S):
        acc = acc + rl[:, s * J:(s + 1) * J] * sph[:, s:s + 1]
    coeff_ref[...] = acc * (1.0 / AVG)


def _h1p_body(nfi_ref, W_ref, out_ref):
    out_ref[...] = jnp.dot(nfi_ref[...], W_ref[...], preferred_element_type=jnp.float32)


def _node_out_body(A_ref, attrs_ref, onehot_ref, nfi_ref, Wg_ref, Wsym_ref,
                   Wh2_ref, Wm2_ref, out_ref):
    attrs = attrs_ref[...]
    linA = []
    for j in range(J):
        Aj = A_ref[j]
        acc = jnp.zeros((BN, C), jnp.float32)
        for z in range(Z):
            t = jnp.dot(Aj, Wg_ref[j, z], preferred_element_type=jnp.float32)
            acc = acc + attrs[:, z:z + 1] * t
        linA.append(acc)
    A0 = linA[0]
    A1sq = linA[1] ** 2 + linA[2] ** 2 + linA[3] ** 2
    feats = [A0, A0 * A0, A1sq, A0 * A0 * A0, A0 * A1sq]
    onehot = onehot_ref[...]
    B = jnp.zeros((BN, C), jnp.float32)
    for z in range(Z):
        oh = onehot[:, z:z + 1]
        for f in range(5):
            B = B + oh * jnp.dot(feats[f], Wsym_ref[z, f], preferred_element_type=jnp.float32)
    out_ref[...] = (jnp.dot(nfi_ref[...], Wh2_ref[...], preferred_element_type=jnp.float32)
                    + jnp.dot(B, Wm2_ref[...], preferred_element_type=jnp.float32))


def _full(shape):
    return pl.BlockSpec(shape, lambda i: tuple(0 for _ in shape))


def kernel(sender, receiver, indices, node_attrs, node_feature_in,
           edge_length_embed, edge_sph_embed,
           W_h1p, W0, b0, W1, b1, W2, b2, W3, b3, W_mix, Wsym, W_h2, W_m2):
    f32 = jnp.float32

    coeff = pl.pallas_call(
        _edge_coeff_body,
        grid=(E // BE,),
        in_specs=[
            pl.BlockSpec((BE, R), lambda i: (i, 0)),
            pl.BlockSpec((BE, S), lambda i: (i, 0)),
            _full((R, 32)), _full((32,)),
            _full((32, 32)), _full((32,)),
            _full((32, 32)), _full((32,)),
            _full((32, S * J)), _full((S * J,)),
        ],
        out_specs=pl.BlockSpec((BE, J), lambda i: (i, 0)),
        out_shape=jax.ShapeDtypeStruct((E, J), f32),
    )(edge_length_embed, edge_sph_embed, W0, b0, W1, b1, W2, b2, W3, b3)

    h1p = pl.pallas_call(
        _h1p_body,
        grid=(N // BN,),
        in_specs=[pl.BlockSpec((BN, C), lambda i: (i, 0)), _full((C, C))],
        out_specs=pl.BlockSpec((BN, C), lambda i: (i, 0)),
        out_shape=jax.ShapeDtypeStruct((N, C), f32),
    )(node_feature_in, W_h1p)

    hs = jnp.take(h1p, sender, axis=0)
    msg = hs[:, None, :] * coeff[:, :, None]
    A = jax.ops.segment_sum(msg.reshape(E, J * C), receiver, num_segments=N)
    A = A.reshape(N, J, C).transpose(1, 0, 2)

    comp2ir = jnp.array([0, 1, 1, 1])
    Wg = jnp.transpose(W_mix[comp2ir], (0, 3, 1, 2))
    Wsym_t = jnp.transpose(Wsym, (0, 3, 1, 2))
    onehot = jax.nn.one_hot(indices, Z, dtype=f32)

    out = pl.pallas_call(
        _node_out_body,
        grid=(N // BN,),
        in_specs=[
            pl.BlockSpec((J, BN, C), lambda i: (0, i, 0)),
            pl.BlockSpec((BN, Z), lambda i: (i, 0)),
            pl.BlockSpec((BN, Z), lambda i: (i, 0)),
            pl.BlockSpec((BN, C), lambda i: (i, 0)),
            _full((J, Z, C, C)),
            _full((Z, 5, C, C)),
            _full((C, C)), _full((C, C)),
        ],
        out_specs=pl.BlockSpec((BN, C), lambda i: (i, 0)),
        out_shape=jax.ShapeDtypeStruct((N, C), f32),
    )(A, node_attrs, onehot, node_feature_in, Wg, Wsym_t, W_h2, W_m2)
    return out


# speedup vs baseline: 1.9336x; 1.9336x over previous
import jax
import jax.numpy as jnp
from jax import lax
from jax.experimental import pallas as pl
from jax.experimental.pallas import tpu as pltpu
from jax.experimental.pallas import tpu_sc as plsc

N = 10000
E = 160000
C = 128
Z = 4
S = 9
J = 4
R = 8
AVG = 16.0

BE = 2000
BN = 2000

NS = 16
B = 128
RPT = 80
EPAD = NS * B * RPT
ROWS = EPAD // B
NPAD = 10240
NPT = NPAD // NS
OCH = 128
ZR = 32
CH = 16


def _edge_coeff_body(elen_ref, sph_ref, W0_ref, b0_ref, W1_ref, b1_ref,
                     W2_ref, b2_ref, W3_ref, b3_ref, coeff_ref):
    x = elen_ref[...]
    x = jax.nn.silu(jnp.dot(x, W0_ref[...], preferred_element_type=jnp.float32) + b0_ref[...])
    x = x + jax.nn.silu(jnp.dot(x, W1_ref[...], preferred_element_type=jnp.float32) + b1_ref[...])
    x = x + jax.nn.silu(jnp.dot(x, W2_ref[...], preferred_element_type=jnp.float32) + b2_ref[...])
    rl = jnp.dot(x, W3_ref[...], preferred_element_type=jnp.float32) + b3_ref[...]
    sph = sph_ref[...]
    acc = jnp.zeros((BE, J), jnp.float32)
    for s in range(S):
        acc = acc + rl[:, s * J:(s + 1) * J] * sph[:, s:s + 1]
    coeff_ref[...] = acc * (1.0 / AVG)


def _h1p_body(nfi_ref, W_ref, out_ref):
    out_ref[...] = jnp.dot(nfi_ref[...], W_ref[...], preferred_element_type=jnp.float32)


def _node_out_body(A_ref, attrs_ref, onehot_ref, nfi_ref, Wg_ref, Wsym_ref,
                   Wh2_ref, Wm2_ref, out_ref):
    attrs = attrs_ref[...]
    linA = []
    for j in range(J):
        Aj = A_ref[j]
        acc = jnp.zeros((BN, C), jnp.float32)
        for z in range(Z):
            t = jnp.dot(Aj, Wg_ref[j, z], preferred_element_type=jnp.float32)
            acc = acc + attrs[:, z:z + 1] * t
        linA.append(acc)
    A0 = linA[0]
    A1sq = linA[1] ** 2 + linA[2] ** 2 + linA[3] ** 2
    feats = [A0, A0 * A0, A1sq, A0 * A0 * A0, A0 * A1sq]
    onehot = onehot_ref[...]
    Bacc = jnp.zeros((BN, C), jnp.float32)
    for z in range(Z):
        oh = onehot[:, z:z + 1]
        for f in range(5):
            Bacc = Bacc + oh * jnp.dot(feats[f], Wsym_ref[z, f], preferred_element_type=jnp.float32)
    out_ref[...] = (jnp.dot(nfi_ref[...], Wh2_ref[...], preferred_element_type=jnp.float32)
                    + jnp.dot(Bacc, Wm2_ref[...], preferred_element_type=jnp.float32))


def _sc_segment_body(h1p_hbm, send_hbm, recv_hbm, cft_hbm, out_hbm,
                     send_v, recv_v, cf_v, hb0, hb1, zbuf, A_sh,
                     gs0, gs1, ss0, ss1):
    hbufs = (hb0, hb1)
    gsems = (gs0, gs1)
    ssems = (ss0, ss1)
    cid = lax.axis_index("c")
    sid = lax.axis_index("s")
    r0 = sid * RPT

    def _zrow(i, carry):
        for m in range(C // 16):
            zbuf[i, pl.ds(16 * m, 16)] = jnp.zeros((16,), jnp.float32)
        return carry
    lax.fori_loop(0, ZR, _zrow, 0)

    def issue_gather(b, k):
        pltpu.async_copy(h1p_hbm.at[send_v.at[b]], hbufs[k], gsems[k])

    def wait_gather(k):
        pltpu.make_async_copy(h1p_hbm.at[send_v.at[0]], hbufs[k], gsems[k]).wait()

    def issue_scatter(b, k):
        pltpu.async_copy(hbufs[k], A_sh.at[recv_v.at[b]], ssems[k], add=True)

    def wait_scatter(k):
        pltpu.make_async_copy(hbufs[k], A_sh.at[recv_v.at[0]], ssems[k]).wait()

    def process(b, k):
        wait_gather(k)
        cf_row = cf_v.at[b]
        hb = hbufs[k]

        def _group(g, carry):
            cvec = cf_row[pl.ds(16 * g, 16)]
            for rr in range(16):
                e = 16 * g + rr
                splat = jnp.take_along_axis(
                    cvec, jnp.full((16,), rr, jnp.int32), axis=0,
                    mode="promise_in_bounds")
                for m in range(C // 16):
                    sl = pl.ds(16 * m, 16)
                    hb[e, sl] = hb[e, sl] * splat
            return carry
        lax.fori_loop(0, B // 16, _group, 0)

        issue_scatter(b, k)
        bn = b + 2

        @pl.when(bn <= CH - 1)
        def _():
            wait_scatter(k)
            issue_gather(bn, k)

    for p in range(2):
        jj = cid * 2 + p
        for ch in range(NPT // ZR):
            rr = sid * NPT + ch * ZR
            pltpu.sync_copy(zbuf, A_sh.at[pl.ds(rr, ZR)])
        plsc.subcore_barrier()

        for chunk in range(RPT // CH):
            rbase = r0 + chunk * CH
            pltpu.sync_copy(send_hbm.at[pl.ds(rbase, CH)], send_v)
            pltpu.sync_copy(recv_hbm.at[pl.ds(rbase, CH)], recv_v)
            pltpu.sync_copy(cft_hbm.at[jj].at[pl.ds(rbase, CH)], cf_v)

            issue_gather(0, 0)
            issue_gather(1, 1)

            def _ring(i, carry):
                process(2 * i, 0)
                process(2 * i + 1, 1)
                return carry
            lax.fori_loop(0, CH // 2, _ring, 0)
            wait_scatter(0)
            wait_scatter(1)
        plsc.subcore_barrier()

        for ch in range(NPT // OCH):
            rr = sid * NPT + ch * OCH
            pltpu.sync_copy(A_sh.at[pl.ds(rr, OCH)], hb0)
            pltpu.sync_copy(hb0, out_hbm.at[jj].at[pl.ds(rr, OCH)])
        plsc.subcore_barrier()


def _full(shape):
    return pl.BlockSpec(shape, lambda i: tuple(0 for _ in shape))


def kernel(sender, receiver, indices, node_attrs, node_feature_in,
           edge_length_embed, edge_sph_embed,
           W_h1p, W0, b0, W1, b1, W2, b2, W3, b3, W_mix, Wsym, W_h2, W_m2):
    f32 = jnp.float32

    coeff = pl.pallas_call(
        _edge_coeff_body,
        grid=(E // BE,),
        in_specs=[
            pl.BlockSpec((BE, R), lambda i: (i, 0)),
            pl.BlockSpec((BE, S), lambda i: (i, 0)),
            _full((R, 32)), _full((32,)),
            _full((32, 32)), _full((32,)),
            _full((32, 32)), _full((32,)),
            _full((32, S * J)), _full((S * J,)),
        ],
        out_specs=pl.BlockSpec((BE, J), lambda i: (i, 0)),
        out_shape=jax.ShapeDtypeStruct((E, J), f32),
    )(edge_length_embed, edge_sph_embed, W0, b0, W1, b1, W2, b2, W3, b3)

    h1p = pl.pallas_call(
        _h1p_body,
        grid=(N // BN,),
        in_specs=[pl.BlockSpec((BN, C), lambda i: (i, 0)), _full((C, C))],
        out_specs=pl.BlockSpec((BN, C), lambda i: (i, 0)),
        out_shape=jax.ShapeDtypeStruct((N, C), f32),
    )(node_feature_in, W_h1p)

    pad = EPAD - E
    send_r = jnp.pad(sender.astype(jnp.int32), (0, pad)).reshape(ROWS, B)
    recv_r = jnp.pad(receiver.astype(jnp.int32), (0, pad)).reshape(ROWS, B)
    cft_r = jnp.pad(coeff.T, ((0, 0), (0, pad))).reshape(J, ROWS, B)

    mesh = plsc.VectorSubcoreMesh(core_axis_name="c", subcore_axis_name="s")
    A = pl.kernel(
        _sc_segment_body,
        out_type=jax.ShapeDtypeStruct((J, NPAD, C), f32),
        mesh=mesh,
        scratch_types=[
            pltpu.VMEM((CH, B), jnp.int32),
            pltpu.VMEM((CH, B), jnp.int32),
            pltpu.VMEM((CH, B), f32),
            pltpu.VMEM((B, C), f32),
            pltpu.VMEM((B, C), f32),
            pltpu.VMEM((ZR, C), f32),
            pltpu.VMEM_SHARED((NPAD, C), f32),
            pltpu.SemaphoreType.DMA, pltpu.SemaphoreType.DMA,
            pltpu.SemaphoreType.DMA, pltpu.SemaphoreType.DMA,
        ],
    )(h1p, send_r, recv_r, cft_r)
    A = A[:, :N, :]

    comp2ir = jnp.array([0, 1, 1, 1])
    Wg = jnp.transpose(W_mix[comp2ir], (0, 3, 1, 2))
    Wsym_t = jnp.transpose(Wsym, (0, 3, 1, 2))
    onehot = jax.nn.one_hot(indices, Z, dtype=f32)

    out = pl.pallas_call(
        _node_out_body,
        grid=(N // BN,),
        in_specs=[
            pl.BlockSpec((J, BN, C), lambda i: (0, i, 0)),
            pl.BlockSpec((BN, Z), lambda i: (i, 0)),
            pl.BlockSpec((BN, Z), lambda i: (i, 0)),
            pl.BlockSpec((BN, C), lambda i: (i, 0)),
            _full((J, Z, C, C)),
            _full((Z, 5, C, C)),
            _full((C, C)), _full((C, C)),
        ],
        out_specs=pl.BlockSpec((BN, C), lambda i: (i, 0)),
        out_shape=jax.ShapeDtypeStruct((N, C), f32),
    )(A, node_attrs, onehot, node_feature_in, Wg, Wsym_t, W_h2, W_m2)
    return out


# speedup vs baseline: 2.3972x; 1.2398x over previous
import jax
import jax.numpy as jnp
from jax import lax
from jax.experimental import pallas as pl
from jax.experimental.pallas import tpu as pltpu
from jax.experimental.pallas import tpu_sc as plsc

N = 10000
E = 160000
C = 128
Z = 4
S = 9
J = 4
R = 8
AVG = 16.0

BE = 2000
BN = 2000

NS = 16
B = 128
RPT = 80
EPAD = NS * B * RPT
ROWS = EPAD // B
NPAD = 10240
NPT = NPAD // NS
OCH = 128
ZR = 16
CH = 16
DUMP = NPAD - 1


def _edge_coeff_body(elen_ref, sph_ref, W0_ref, b0_ref, W1_ref, b1_ref,
                     W2_ref, b2_ref, W3_ref, b3_ref, K_ref, M_ref, coeff_ref):
    x = elen_ref[...]
    x = jax.nn.silu(jnp.dot(x, W0_ref[...], preferred_element_type=jnp.float32) + b0_ref[...])
    x = x + jax.nn.silu(jnp.dot(x, W1_ref[...], preferred_element_type=jnp.float32) + b1_ref[...])
    x = x + jax.nn.silu(jnp.dot(x, W2_ref[...], preferred_element_type=jnp.float32) + b2_ref[...])
    rl = jnp.dot(x, W3_ref[...], preferred_element_type=jnp.float32) + b3_ref[...]
    rep = jnp.dot(sph_ref[...], K_ref[...], preferred_element_type=jnp.float32)
    coeff_ref[...] = jnp.dot(rl * rep, M_ref[...], preferred_element_type=jnp.float32)


def _h1p_body(nfi_ref, W_ref, out_ref):
    out_ref[...] = jnp.dot(nfi_ref[...], W_ref[...], preferred_element_type=jnp.float32)


def _node_out_body(A_ref, attrs_ref, onehot_ref, nfi_ref, Wg_ref, Wsym_ref,
                   Wh2_ref, Wm2_ref, out_ref):
    attrs = attrs_ref[...]
    linA = []
    for j in range(J):
        Aj = A_ref[j]
        acc = jnp.zeros((BN, C), jnp.float32)
        for z in range(Z):
            t = jnp.dot(Aj, Wg_ref[j, z], preferred_element_type=jnp.float32)
            acc = acc + attrs[:, z:z + 1] * t
        linA.append(acc)
    A0 = linA[0]
    A1sq = linA[1] ** 2 + linA[2] ** 2 + linA[3] ** 2
    feats = [A0, A0 * A0, A1sq, A0 * A0 * A0, A0 * A1sq]
    onehot = onehot_ref[...]
    Bacc = jnp.zeros((BN, C), jnp.float32)
    for z in range(Z):
        oh = onehot[:, z:z + 1]
        for f in range(5):
            Bacc = Bacc + oh * jnp.dot(feats[f], Wsym_ref[z, f], preferred_element_type=jnp.float32)
    out_ref[...] = (jnp.dot(nfi_ref[...], Wh2_ref[...], preferred_element_type=jnp.float32)
                    + jnp.dot(Bacc, Wm2_ref[...], preferred_element_type=jnp.float32))


def _sc_segment_body(h1p_hbm, send_hbm, recv_hbm, cft_hbm, out_hbm,
                     send_v, recv_v, cf_v, hb0, hb1, zbuf, A_sh,
                     gs0, gs1, ss0, ss1):
    hbufs = (hb0, hb1)
    gsems = (gs0, gs1)
    ssems = (ss0, ss1)
    cid = lax.axis_index("c")
    sid = lax.axis_index("s")
    r0 = sid * RPT

    def _zrow(i, carry):
        for m in range(C // 16):
            zbuf[i, pl.ds(16 * m, 16)] = jnp.zeros((16,), jnp.float32)
        return carry
    lax.fori_loop(0, ZR, _zrow, 0)

    def issue_gather(b, k):
        pltpu.async_copy(h1p_hbm.at[send_v.at[b]], hbufs[k], gsems[k])

    def wait_gather(k):
        pltpu.make_async_copy(h1p_hbm.at[send_v.at[0]], hbufs[k], gsems[k]).wait()

    def issue_scatter(b, k):
        pltpu.async_copy(hbufs[k], A_sh.at[recv_v.at[b]], ssems[k], add=True)

    def wait_scatter(k):
        pltpu.make_async_copy(hbufs[k], A_sh.at[recv_v.at[0]], ssems[k]).wait()

    def process(b, k, jj):
        wait_gather(k)
        cf_row = cf_v.at[b]
        hb = hbufs[k]

        def _group(g, carry):
            cvec = cf_row[pl.ds(16 * g, 16)]
            for r4 in range(4):
                e = 4 * g + r4
                splat = jnp.take_along_axis(
                    cvec, jnp.full((16,), J * r4, jnp.int32) + jj, axis=0,
                    mode="promise_in_bounds")
                for m in range(C // 16):
                    sl = pl.ds(16 * m, 16)
                    hb[e, sl] = hb[e, sl] * splat
            return carry
        lax.fori_loop(0, B // 4, _group, 0)

        issue_scatter(b, k)
        bn = b + 2

        @pl.when(bn <= CH - 1)
        def _():
            wait_scatter(k)
            issue_gather(bn, k)

    for p in range(2):
        jj = cid * 2 + p
        for ch in range(NPT // ZR):
            rr = sid * NPT + ch * ZR
            pltpu.sync_copy(zbuf, A_sh.at[pl.ds(rr, ZR)])
        plsc.subcore_barrier()

        for chunk in range(RPT // CH):
            rbase = r0 + chunk * CH
            pltpu.sync_copy(send_hbm.at[pl.ds(rbase, CH)], send_v)
            pltpu.sync_copy(recv_hbm.at[pl.ds(rbase, CH)], recv_v)
            pltpu.sync_copy(cft_hbm.at[pl.ds(rbase, CH)], cf_v)

            issue_gather(0, 0)
            issue_gather(1, 1)

            def _ring(i, carry):
                process(2 * i, 0, jj)
                process(2 * i + 1, 1, jj)
                return carry
            lax.fori_loop(0, CH // 2, _ring, 0)
            wait_scatter(0)
            wait_scatter(1)
        plsc.subcore_barrier()

        for ch in range(NPT // OCH):
            rr = sid * NPT + ch * OCH
            pltpu.sync_copy(A_sh.at[pl.ds(rr, OCH)], hb0)
            pltpu.sync_copy(hb0, out_hbm.at[jj].at[pl.ds(rr, OCH)])
        plsc.subcore_barrier()


def _full(shape):
    return pl.BlockSpec(shape, lambda i: tuple(0 for _ in shape))


def kernel(sender, receiver, indices, node_attrs, node_feature_in,
           edge_length_embed, edge_sph_embed,
           W_h1p, W0, b0, W1, b1, W2, b2, W3, b3, W_mix, Wsym, W_h2, W_m2):
    f32 = jnp.float32

    Kmat = jnp.repeat(jnp.eye(S, dtype=f32), J, axis=1)
    Mmat = jnp.tile(jnp.eye(J, dtype=f32), (S, 1)) * (1.0 / AVG)
    coeff = pl.pallas_call(
        _edge_coeff_body,
        grid=(E // BE,),
        in_specs=[
            pl.BlockSpec((BE, R), lambda i: (i, 0)),
            pl.BlockSpec((BE, S), lambda i: (i, 0)),
            _full((R, 32)), _full((32,)),
            _full((32, 32)), _full((32,)),
            _full((32, 32)), _full((32,)),
            _full((32, S * J)), _full((S * J,)),
            _full((S, S * J)), _full((S * J, J)),
        ],
        out_specs=pl.BlockSpec((BE, J), lambda i: (i, 0)),
        out_shape=jax.ShapeDtypeStruct((EPAD, J), f32),
    )(edge_length_embed, edge_sph_embed, W0, b0, W1, b1, W2, b2, W3, b3,
      Kmat, Mmat)

    h1p = pl.pallas_call(
        _h1p_body,
        grid=(N // BN,),
        in_specs=[pl.BlockSpec((BN, C), lambda i: (i, 0)), _full((C, C))],
        out_specs=pl.BlockSpec((BN, C), lambda i: (i, 0)),
        out_shape=jax.ShapeDtypeStruct((N, C), f32),
    )(node_feature_in, W_h1p)

    pad = EPAD - E
    send_r = jnp.pad(sender.astype(jnp.int32), (0, pad)).reshape(ROWS, B)
    recv_r = jnp.pad(receiver.astype(jnp.int32), (0, pad),
                     constant_values=DUMP).reshape(ROWS, B)
    cft_r = coeff.reshape(ROWS, B * J)

    mesh = plsc.VectorSubcoreMesh(core_axis_name="c", subcore_axis_name="s")
    A = pl.kernel(
        _sc_segment_body,
        out_type=jax.ShapeDtypeStruct((J, NPAD, C), f32),
        mesh=mesh,
        scratch_types=[
            pltpu.VMEM((CH, B), jnp.int32),
            pltpu.VMEM((CH, B), jnp.int32),
            pltpu.VMEM((CH, B * J), f32),
            pltpu.VMEM((B, C), f32),
            pltpu.VMEM((B, C), f32),
            pltpu.VMEM((ZR, C), f32),
            pltpu.VMEM_SHARED((NPAD, C), f32),
            pltpu.SemaphoreType.DMA, pltpu.SemaphoreType.DMA,
            pltpu.SemaphoreType.DMA, pltpu.SemaphoreType.DMA,
        ],
    )(h1p, send_r, recv_r, cft_r)
    A = A[:, :N, :]

    comp2ir = jnp.array([0, 1, 1, 1])
    Wg = jnp.transpose(W_mix[comp2ir], (0, 3, 1, 2))
    Wsym_t = jnp.transpose(Wsym, (0, 3, 1, 2))
    onehot = jax.nn.one_hot(indices, Z, dtype=f32)

    out = pl.pallas_call(
        _node_out_body,
        grid=(N // BN,),
        in_specs=[
            pl.BlockSpec((J, BN, C), lambda i: (0, i, 0)),
            pl.BlockSpec((BN, Z), lambda i: (i, 0)),
            pl.BlockSpec((BN, Z), lambda i: (i, 0)),
            pl.BlockSpec((BN, C), lambda i: (i, 0)),
            _full((J, Z, C, C)),
            _full((Z, 5, C, C)),
            _full((C, C)), _full((C, C)),
        ],
        out_specs=pl.BlockSpec((BN, C), lambda i: (i, 0)),
        out_shape=jax.ShapeDtypeStruct((N, C), f32),
    )(A, node_attrs, onehot, node_feature_in, Wg, Wsym_t, W_h2, W_m2)
    return out


# speedup vs baseline: 2.4124x; 1.0064x over previous
import jax
import jax.numpy as jnp
from jax import lax
from jax.experimental import pallas as pl
from jax.experimental.pallas import tpu as pltpu
from jax.experimental.pallas import tpu_sc as plsc

N = 10000
E = 160000
C = 128
Z = 4
S = 9
J = 4
R = 8
AVG = 16.0

BE = 2000
BN = 2000

NS = 16
B = 128
RPT = 80
EPAD = NS * B * RPT
ROWS = EPAD // B
NPAD = 10240
NPT = NPAD // NS
OCH = 128
ZR = 16
CH = 16
DUMP = NPAD - 1


def _edge_coeff_body(elen_ref, sph_ref, W0_ref, b0_ref, W1_ref, b1_ref,
                     W2_ref, b2_ref, W3_ref, b3_ref, K_ref, M_ref, coeff_ref):
    x = elen_ref[...]
    x = jax.nn.silu(jnp.dot(x, W0_ref[...], preferred_element_type=jnp.float32) + b0_ref[...])
    x = x + jax.nn.silu(jnp.dot(x, W1_ref[...], preferred_element_type=jnp.float32) + b1_ref[...])
    x = x + jax.nn.silu(jnp.dot(x, W2_ref[...], preferred_element_type=jnp.float32) + b2_ref[...])
    rl = jnp.dot(x, W3_ref[...], preferred_element_type=jnp.float32) + b3_ref[...]
    rep = jnp.dot(sph_ref[...], K_ref[...], preferred_element_type=jnp.float32)
    coeff_ref[...] = jnp.dot(rl * rep, M_ref[...], preferred_element_type=jnp.float32)


def _h1p_body(nfi_ref, W_ref, out_ref):
    out_ref[...] = jnp.dot(nfi_ref[...], W_ref[...], preferred_element_type=jnp.float32)


def _node_out_body(A_ref, attrs_ref, onehot_ref, nfi_ref, Wg_ref, Wsym_ref,
                   Wh2_ref, Wm2_ref, out_ref):
    attrs = attrs_ref[...]
    linA = []
    for j in range(J):
        t = jnp.dot(A_ref[j], Wg_ref[j], preferred_element_type=jnp.float32)
        acc = jnp.zeros((BN, C), jnp.float32)
        for z in range(Z):
            acc = acc + attrs[:, z:z + 1] * t[:, z * C:(z + 1) * C]
        linA.append(acc)
    A0 = linA[0]
    A1sq = linA[1] ** 2 + linA[2] ** 2 + linA[3] ** 2
    feats = jnp.concatenate(
        [A0, A0 * A0, A1sq, A0 * A0 * A0, A0 * A1sq], axis=1)
    res = jnp.dot(feats, Wsym_ref[...], preferred_element_type=jnp.float32)
    onehot = onehot_ref[...]
    Bacc = jnp.zeros((BN, C), jnp.float32)
    for z in range(Z):
        Bacc = Bacc + onehot[:, z:z + 1] * res[:, z * C:(z + 1) * C]
    out_ref[...] = (jnp.dot(nfi_ref[...], Wh2_ref[...], preferred_element_type=jnp.float32)
                    + jnp.dot(Bacc, Wm2_ref[...], preferred_element_type=jnp.float32))


def _sc_segment_body(h1p_hbm, send_hbm, recv_hbm, cft_hbm, out_hbm,
                     send_v, recv_v, cf_v, hb0, hb1, zbuf, A_sh,
                     gs0, gs1, ss0, ss1):
    hbufs = (hb0, hb1)
    gsems = (gs0, gs1)
    ssems = (ss0, ss1)
    cid = lax.axis_index("c")
    sid = lax.axis_index("s")
    r0 = sid * RPT

    def _zrow(i, carry):
        for m in range(C // 16):
            zbuf[i, pl.ds(16 * m, 16)] = jnp.zeros((16,), jnp.float32)
        return carry
    lax.fori_loop(0, ZR, _zrow, 0)

    def issue_gather(b, k):
        pltpu.async_copy(h1p_hbm.at[send_v.at[b]], hbufs[k], gsems[k])

    def wait_gather(k):
        pltpu.make_async_copy(h1p_hbm.at[send_v.at[0]], hbufs[k], gsems[k]).wait()

    def issue_scatter(b, k):
        pltpu.async_copy(hbufs[k], A_sh.at[recv_v.at[b]], ssems[k], add=True)

    def wait_scatter(k):
        pltpu.make_async_copy(hbufs[k], A_sh.at[recv_v.at[0]], ssems[k]).wait()

    def process(b, k, jj):
        wait_gather(k)
        cf_row = cf_v.at[b]
        hb = hbufs[k]

        def _group(g, carry):
            cvec = cf_row[pl.ds(16 * g, 16)]
            for r4 in range(4):
                e = 4 * g + r4
                splat = jnp.take_along_axis(
                    cvec, jnp.full((16,), J * r4, jnp.int32) + jj, axis=0,
                    mode="promise_in_bounds")
                for m in range(C // 16):
                    sl = pl.ds(16 * m, 16)
                    hb[e, sl] = hb[e, sl] * splat
            return carry
        lax.fori_loop(0, B // 4, _group, 0)

        issue_scatter(b, k)
        bn = b + 2

        @pl.when(bn <= CH - 1)
        def _():
            wait_scatter(k)
            issue_gather(bn, k)

    for p in range(2):
        jj = cid * 2 + p
        for ch in range(NPT // ZR):
            rr = sid * NPT + ch * ZR
            pltpu.sync_copy(zbuf, A_sh.at[pl.ds(rr, ZR)])
        plsc.subcore_barrier()

        for chunk in range(RPT // CH):
            rbase = r0 + chunk * CH
            pltpu.sync_copy(send_hbm.at[pl.ds(rbase, CH)], send_v)
            pltpu.sync_copy(recv_hbm.at[pl.ds(rbase, CH)], recv_v)
            pltpu.sync_copy(cft_hbm.at[pl.ds(rbase, CH)], cf_v)

            issue_gather(0, 0)
            issue_gather(1, 1)

            def _ring(i, carry):
                process(2 * i, 0, jj)
                process(2 * i + 1, 1, jj)
                return carry
            lax.fori_loop(0, CH // 2, _ring, 0)
            wait_scatter(0)
            wait_scatter(1)
        plsc.subcore_barrier()

        for ch in range(NPT // OCH):
            rr = sid * NPT + ch * OCH
            pltpu.sync_copy(A_sh.at[pl.ds(rr, OCH)], hb0)
            pltpu.sync_copy(hb0, out_hbm.at[jj].at[pl.ds(rr, OCH)])
        plsc.subcore_barrier()


def _full(shape):
    return pl.BlockSpec(shape, lambda i: tuple(0 for _ in shape))


def kernel(sender, receiver, indices, node_attrs, node_feature_in,
           edge_length_embed, edge_sph_embed,
           W_h1p, W0, b0, W1, b1, W2, b2, W3, b3, W_mix, Wsym, W_h2, W_m2):
    f32 = jnp.float32

    Kmat = jnp.repeat(jnp.eye(S, dtype=f32), J, axis=1)
    Mmat = jnp.tile(jnp.eye(J, dtype=f32), (S, 1)) * (1.0 / AVG)
    coeff = pl.pallas_call(
        _edge_coeff_body,
        grid=(E // BE,),
        in_specs=[
            pl.BlockSpec((BE, R), lambda i: (i, 0)),
            pl.BlockSpec((BE, S), lambda i: (i, 0)),
            _full((R, 32)), _full((32,)),
            _full((32, 32)), _full((32,)),
            _full((32, 32)), _full((32,)),
            _full((32, S * J)), _full((S * J,)),
            _full((S, S * J)), _full((S * J, J)),
        ],
        out_specs=pl.BlockSpec((BE, J), lambda i: (i, 0)),
        out_shape=jax.ShapeDtypeStruct((EPAD, J), f32),
    )(edge_length_embed, edge_sph_embed, W0, b0, W1, b1, W2, b2, W3, b3,
      Kmat, Mmat)

    h1p = pl.pallas_call(
        _h1p_body,
        grid=(N // BN,),
        in_specs=[pl.BlockSpec((BN, C), lambda i: (i, 0)), _full((C, C))],
        out_specs=pl.BlockSpec((BN, C), lambda i: (i, 0)),
        out_shape=jax.ShapeDtypeStruct((N, C), f32),
    )(node_feature_in, W_h1p)

    pad = EPAD - E
    send_r = jnp.pad(sender.astype(jnp.int32), (0, pad)).reshape(ROWS, B)
    recv_r = jnp.pad(receiver.astype(jnp.int32), (0, pad),
                     constant_values=DUMP).reshape(ROWS, B)
    cft_r = coeff.reshape(ROWS, B * J)

    mesh = plsc.VectorSubcoreMesh(core_axis_name="c", subcore_axis_name="s")
    A = pl.kernel(
        _sc_segment_body,
        out_type=jax.ShapeDtypeStruct((J, NPAD, C), f32),
        mesh=mesh,
        scratch_types=[
            pltpu.VMEM((CH, B), jnp.int32),
            pltpu.VMEM((CH, B), jnp.int32),
            pltpu.VMEM((CH, B * J), f32),
            pltpu.VMEM((B, C), f32),
            pltpu.VMEM((B, C), f32),
            pltpu.VMEM((ZR, C), f32),
            pltpu.VMEM_SHARED((NPAD, C), f32),
            pltpu.SemaphoreType.DMA, pltpu.SemaphoreType.DMA,
            pltpu.SemaphoreType.DMA, pltpu.SemaphoreType.DMA,
        ],
    )(h1p, send_r, recv_r, cft_r)
    A = A[:, :N, :]

    comp2ir = jnp.array([0, 1, 1, 1])
    Wg = jnp.transpose(W_mix[comp2ir], (0, 1, 3, 2)).reshape(J, C, Z * C)
    Wsym_t = jnp.transpose(Wsym, (3, 1, 0, 2)).reshape(5 * C, Z * C)
    onehot = jax.nn.one_hot(indices, Z, dtype=f32)

    out = pl.pallas_call(
        _node_out_body,
        grid=(N // BN,),
        in_specs=[
            pl.BlockSpec((J, BN, C), lambda i: (0, i, 0)),
            pl.BlockSpec((BN, Z), lambda i: (i, 0)),
            pl.BlockSpec((BN, Z), lambda i: (i, 0)),
            pl.BlockSpec((BN, C), lambda i: (i, 0)),
            _full((J, C, Z * C)),
            _full((5 * C, Z * C)),
            _full((C, C)), _full((C, C)),
        ],
        out_specs=pl.BlockSpec((BN, C), lambda i: (i, 0)),
        out_shape=jax.ShapeDtypeStruct((N, C), f32),
    )(A, node_attrs, onehot, node_feature_in, Wg, Wsym_t, W_h2, W_m2)
    return out


# speedup vs baseline: 4.2978x; 1.7816x over previous
import jax
import jax.numpy as jnp
from jax import lax
from jax.experimental import pallas as pl
from jax.experimental.pallas import tpu as pltpu
from jax.experimental.pallas import tpu_sc as plsc

N = 10000
E = 160000
C = 128
Z = 4
S = 9
J = 4
R = 8
AVG = 16.0

BE = 2000
BN = 2000

NS = 16
B = 128
RPT = 80
EPAD = NS * B * RPT
ROWS = EPAD // B
NPAD = 10240
NPT = NPAD // NS
OCH = 128
ZR = 16
CH = 16
DUMP = NPAD - 1


def _edge_coeff_body(elen_ref, sph_ref, W0_ref, b0_ref, W1_ref, b1_ref,
                     W2_ref, b2_ref, W3_ref, b3_ref, K_ref, M_ref, coeff_ref):
    x = elen_ref[...]
    x = jax.nn.silu(jnp.dot(x, W0_ref[...], preferred_element_type=jnp.float32) + b0_ref[...])
    x = x + jax.nn.silu(jnp.dot(x, W1_ref[...], preferred_element_type=jnp.float32) + b1_ref[...])
    x = x + jax.nn.silu(jnp.dot(x, W2_ref[...], preferred_element_type=jnp.float32) + b2_ref[...])
    rl = jnp.dot(x, W3_ref[...], preferred_element_type=jnp.float32) + b3_ref[...]
    rep = jnp.dot(sph_ref[...], K_ref[...], preferred_element_type=jnp.float32)
    coeff_ref[...] = jnp.dot(rl * rep, M_ref[...], preferred_element_type=jnp.float32)


def _h1p_body(nfi_ref, W_ref, out_ref):
    out_ref[...] = jnp.dot(nfi_ref[...], W_ref[...], preferred_element_type=jnp.float32)


def _node_out_body(A_ref, attrs_ref, onehot_ref, nfi_ref, Wg_ref, Wsym_ref,
                   Wh2_ref, Wm2_ref, out_ref):
    attrs = attrs_ref[...]
    linA = []
    for j in range(J):
        t = jnp.dot(A_ref[j], Wg_ref[j], preferred_element_type=jnp.float32)
        acc = jnp.zeros((BN, C), jnp.float32)
        for z in range(Z):
            acc = acc + attrs[:, z:z + 1] * t[:, z * C:(z + 1) * C]
        linA.append(acc)
    A0 = linA[0]
    A1sq = linA[1] ** 2 + linA[2] ** 2 + linA[3] ** 2
    feats = jnp.concatenate(
        [A0, A0 * A0, A1sq, A0 * A0 * A0, A0 * A1sq], axis=1)
    res = jnp.dot(feats, Wsym_ref[...], preferred_element_type=jnp.float32)
    onehot = onehot_ref[...]
    Bacc = jnp.zeros((BN, C), jnp.float32)
    for z in range(Z):
        Bacc = Bacc + onehot[:, z:z + 1] * res[:, z * C:(z + 1) * C]
    out_ref[...] = (jnp.dot(nfi_ref[...], Wh2_ref[...], preferred_element_type=jnp.float32)
                    + jnp.dot(Bacc, Wm2_ref[...], preferred_element_type=jnp.float32))


def _sc_segment_body(h1p_hbm, send_hbm, recv_hbm, cft_hbm, out_hbm,
                     send_v, recv_v, cf_v, hb0, hb1, zbuf, A_sh,
                     gs0, gs1, ss0, ss1):
    hbufs = (hb0, hb1)
    gsems = (gs0, gs1)
    ssems = (ss0, ss1)
    cid = lax.axis_index("c")
    sid = lax.axis_index("s")
    r0 = sid * RPT

    def _zrow(i, carry):
        for m in range(C // 16):
            zbuf[i, pl.ds(16 * m, 16)] = jnp.zeros((16,), jnp.float32)
        return carry
    lax.fori_loop(0, ZR, _zrow, 0)

    def issue_gather(b, k):
        pltpu.async_copy(h1p_hbm.at[send_v.at[b]], hbufs[k], gsems[k])

    def wait_gather(k):
        pltpu.make_async_copy(h1p_hbm.at[send_v.at[0]], hbufs[k], gsems[k]).wait()

    def issue_scatter(b, k):
        pltpu.async_copy(hbufs[k], A_sh.at[recv_v.at[b]], ssems[k], add=True)

    def wait_scatter(k):
        pltpu.make_async_copy(hbufs[k], A_sh.at[recv_v.at[0]], ssems[k]).wait()

    def process(b, k, jj):
        wait_gather(k)
        cf_row = cf_v.at[b]
        hb = hbufs[k]

        def _group(g, carry):
            cvec = cf_row[pl.ds(16 * g, 16)]
            for r4 in range(4):
                e = 4 * g + r4
                splat = jnp.take_along_axis(
                    cvec, jnp.full((16,), J * r4, jnp.int32) + jj, axis=0,
                    mode="promise_in_bounds")
                for m in range(C // 16):
                    sl = pl.ds(16 * m, 16)
                    hb[e, sl] = hb[e, sl] * splat
            return carry
        lax.fori_loop(0, B // 4, _group, 0)

        issue_scatter(b, k)
        bn = b + 2

        @pl.when(bn <= CH - 1)
        def _():
            wait_scatter(k)
            issue_gather(bn, k)

    for p in range(2):
        jj = cid * 2 + p
        for ch in range(NPT // ZR):
            rr = sid * NPT + ch * ZR
            pltpu.sync_copy(zbuf, A_sh.at[pl.ds(rr, ZR)])
        plsc.subcore_barrier()

        for chunk in range(RPT // CH):
            rbase = r0 + chunk * CH
            pltpu.sync_copy(send_hbm.at[pl.ds(rbase, CH)], send_v)
            pltpu.sync_copy(recv_hbm.at[pl.ds(rbase, CH)], recv_v)
            pltpu.sync_copy(cft_hbm.at[pl.ds(rbase, CH)], cf_v)

            issue_gather(0, 0)
            issue_gather(1, 1)

            def _ring(i, carry):
                process(2 * i, 0, jj)
                process(2 * i + 1, 1, jj)
                return carry
            lax.fori_loop(0, CH // 2, _ring, 0)
            wait_scatter(0)
            wait_scatter(1)
        plsc.subcore_barrier()

        for ch in range(NPT // OCH):
            rr = sid * NPT + ch * OCH
            pltpu.sync_copy(A_sh.at[pl.ds(rr, OCH)], hb0)
            pltpu.sync_copy(hb0, out_hbm.at[jj].at[pl.ds(rr, OCH)])
        plsc.subcore_barrier()


def _full(shape):
    return pl.BlockSpec(shape, lambda i: tuple(0 for _ in shape))


def kernel(sender, receiver, indices, node_attrs, node_feature_in,
           edge_length_embed, edge_sph_embed,
           W_h1p, W0, b0, W1, b1, W2, b2, W3, b3, W_mix, Wsym, W_h2, W_m2):
    f32 = jnp.float32

    Kmat = jnp.repeat(jnp.eye(S, dtype=f32), J, axis=1)
    Mmat = jnp.tile(jnp.eye(J, dtype=f32), (S, 1)) * (1.0 / AVG)
    coeff = pl.pallas_call(
        _edge_coeff_body,
        grid=(E // BE,),
        in_specs=[
            pl.BlockSpec((BE, R), lambda i: (i, 0)),
            pl.BlockSpec((BE, S), lambda i: (i, 0)),
            _full((R, 32)), _full((32,)),
            _full((32, 32)), _full((32,)),
            _full((32, 32)), _full((32,)),
            _full((32, S * J)), _full((S * J,)),
            _full((S, S * J)), _full((S * J, J)),
        ],
        out_specs=pl.BlockSpec((BE, J), lambda i: (i, 0)),
        out_shape=jax.ShapeDtypeStruct((EPAD, J), f32),
    )(edge_length_embed, edge_sph_embed, W0, b0, W1, b1, W2, b2, W3, b3,
      Kmat, Mmat)

    h1p = pl.pallas_call(
        _h1p_body,
        grid=(N // BN,),
        in_specs=[pl.BlockSpec((BN, C), lambda i: (i, 0)), _full((C, C))],
        out_specs=pl.BlockSpec((BN, C), lambda i: (i, 0)),
        out_shape=jax.ShapeDtypeStruct((N, C), f32),
    )(node_feature_in, W_h1p)

    pad = EPAD - E
    pad_i = jnp.arange(pad, dtype=jnp.int32)
    send_r = jnp.concatenate(
        [sender.astype(jnp.int32), pad_i % N]).reshape(ROWS, B)
    recv_r = jnp.concatenate(
        [receiver.astype(jnp.int32), N + pad_i % (NPAD - N)]).reshape(ROWS, B)
    cft_r = coeff.reshape(ROWS, B * J)

    mesh = plsc.VectorSubcoreMesh(core_axis_name="c", subcore_axis_name="s")
    A = pl.kernel(
        _sc_segment_body,
        out_type=jax.ShapeDtypeStruct((J, NPAD, C), f32),
        mesh=mesh,
        scratch_types=[
            pltpu.VMEM((CH, B), jnp.int32),
            pltpu.VMEM((CH, B), jnp.int32),
            pltpu.VMEM((CH, B * J), f32),
            pltpu.VMEM((B, C), f32),
            pltpu.VMEM((B, C), f32),
            pltpu.VMEM((ZR, C), f32),
            pltpu.VMEM_SHARED((NPAD, C), f32),
            pltpu.SemaphoreType.DMA, pltpu.SemaphoreType.DMA,
            pltpu.SemaphoreType.DMA, pltpu.SemaphoreType.DMA,
        ],
    )(h1p, send_r, recv_r, cft_r)
    A = A[:, :N, :]

    comp2ir = jnp.array([0, 1, 1, 1])
    Wg = jnp.transpose(W_mix[comp2ir], (0, 1, 3, 2)).reshape(J, C, Z * C)
    Wsym_t = jnp.transpose(Wsym, (3, 1, 0, 2)).reshape(5 * C, Z * C)
    onehot = jax.nn.one_hot(indices, Z, dtype=f32)

    out = pl.pallas_call(
        _node_out_body,
        grid=(N // BN,),
        in_specs=[
            pl.BlockSpec((J, BN, C), lambda i: (0, i, 0)),
            pl.BlockSpec((BN, Z), lambda i: (i, 0)),
            pl.BlockSpec((BN, Z), lambda i: (i, 0)),
            pl.BlockSpec((BN, C), lambda i: (i, 0)),
            _full((J, C, Z * C)),
            _full((5 * C, Z * C)),
            _full((C, C)), _full((C, C)),
        ],
        out_specs=pl.BlockSpec((BN, C), lambda i: (i, 0)),
        out_shape=jax.ShapeDtypeStruct((N, C), f32),
    )(A, node_attrs, onehot, node_feature_in, Wg, Wsym_t, W_h2, W_m2)
    return out


# speedup vs baseline: 4.3982x; 1.0233x over previous
import jax
import jax.numpy as jnp
from jax import lax
from jax.experimental import pallas as pl
from jax.experimental.pallas import tpu as pltpu
from jax.experimental.pallas import tpu_sc as plsc

N = 10000
E = 160000
C = 128
Z = 4
S = 9
J = 4
R = 8
AVG = 16.0

BE = 2000
BN = 2000

NS = 16
B = 128
RPT = 80
EPAD = NS * B * RPT
ROWS = EPAD // B
NPAD = 10240
NPT = NPAD // NS
OCH = 128
ZR = 16
CH = 16
DUMP = NPAD - 1


def _edge_coeff_body(elen_ref, sph_ref, W0_ref, b0_ref, W1_ref, b1_ref,
                     W2_ref, b2_ref, W3_ref, b3_ref, K_ref, M_ref, coeff_ref):
    x = elen_ref[...]
    x = jax.nn.silu(jnp.dot(x, W0_ref[...], preferred_element_type=jnp.float32) + b0_ref[...])
    x = x + jax.nn.silu(jnp.dot(x, W1_ref[...], preferred_element_type=jnp.float32) + b1_ref[...])
    x = x + jax.nn.silu(jnp.dot(x, W2_ref[...], preferred_element_type=jnp.float32) + b2_ref[...])
    rl = jnp.dot(x, W3_ref[...], preferred_element_type=jnp.float32) + b3_ref[...]
    rep = jnp.dot(sph_ref[...], K_ref[...], preferred_element_type=jnp.float32)
    coeff_ref[...] = jnp.dot(rl * rep, M_ref[...], preferred_element_type=jnp.float32)


def _h1p_body(nfi_ref, W_ref, out_ref):
    out_ref[...] = jnp.dot(nfi_ref[...], W_ref[...], preferred_element_type=jnp.float32)


def _node_out_body(A_ref, attrs_ref, onehot_ref, nfi_ref, Wg_ref, Wsym_ref,
                   Wh2_ref, Wm2_ref, out_ref):
    attrs = attrs_ref[...]
    linA = []
    for j in range(J):
        t = jnp.dot(A_ref[j], Wg_ref[j], preferred_element_type=jnp.float32)
        acc = jnp.zeros((BN, C), jnp.float32)
        for z in range(Z):
            acc = acc + attrs[:, z:z + 1] * t[:, z * C:(z + 1) * C]
        linA.append(acc)
    A0 = linA[0]
    A1sq = linA[1] ** 2 + linA[2] ** 2 + linA[3] ** 2
    feats = jnp.concatenate(
        [A0, A0 * A0, A1sq, A0 * A0 * A0, A0 * A1sq], axis=1)
    res = jnp.dot(feats, Wsym_ref[...], preferred_element_type=jnp.float32)
    onehot = onehot_ref[...]
    Bacc = jnp.zeros((BN, C), jnp.float32)
    for z in range(Z):
        Bacc = Bacc + onehot[:, z:z + 1] * res[:, z * C:(z + 1) * C]
    out_ref[...] = (jnp.dot(nfi_ref[...], Wh2_ref[...], preferred_element_type=jnp.float32)
                    + jnp.dot(Bacc, Wm2_ref[...], preferred_element_type=jnp.float32))


def _sc_segment_body(h1p_hbm, send_hbm, recv_hbm, cft_hbm, out_hbm,
                     send_v, recv_v, cf_v, hb0, hb1, zbuf, A_sh,
                     gs0, gs1, ss0, ss1):
    hbufs = (hb0, hb1)
    gsems = (gs0, gs1)
    ssems = (ss0, ss1)
    cid = lax.axis_index("c")
    sid = lax.axis_index("s")
    r0 = sid * RPT

    def _zrow(i, carry):
        for m in range(C // 16):
            zbuf[i, pl.ds(16 * m, 16)] = jnp.zeros((16,), jnp.float32)
        return carry
    lax.fori_loop(0, ZR, _zrow, 0)

    def issue_gather(b, k):
        pltpu.async_copy(h1p_hbm.at[send_v.at[b]], hbufs[k], gsems[k])

    def wait_gather(k):
        pltpu.make_async_copy(h1p_hbm.at[send_v.at[0]], hbufs[k], gsems[k]).wait()

    def issue_scatter(b, k):
        pltpu.async_copy(hbufs[k], A_sh.at[recv_v.at[b]], ssems[k], add=True)

    def wait_scatter(k):
        pltpu.make_async_copy(hbufs[k], A_sh.at[recv_v.at[0]], ssems[k]).wait()

    def process(b, k, jj):
        wait_gather(k)
        cf_row = cf_v.at[b]
        hb = hbufs[k]

        def _group(g, carry):
            cvec = cf_row[pl.ds(16 * g, 16)]
            for r4 in range(4):
                e = 4 * g + r4
                splat = jnp.take_along_axis(
                    cvec, jnp.full((16,), J * r4, jnp.int32) + jj, axis=0,
                    mode="promise_in_bounds")
                for m in range(C // 16):
                    sl = pl.ds(16 * m, 16)
                    hb[e, sl] = hb[e, sl] * splat
            return carry
        lax.fori_loop(0, B // 4, _group, 0)

        issue_scatter(b, k)
        bn = b + 2

        @pl.when(bn <= CH - 1)
        def _():
            wait_scatter(k)
            issue_gather(bn, k)

    for p in range(2):
        jj = cid * 2 + p
        for ch in range(NPT // ZR):
            rr = sid * NPT + ch * ZR
            pltpu.sync_copy(zbuf, A_sh.at[pl.ds(rr, ZR)])
        plsc.subcore_barrier()

        for chunk in range(RPT // CH):
            rbase = r0 + chunk * CH
            pltpu.sync_copy(send_hbm.at[pl.ds(rbase, CH)], send_v)
            pltpu.sync_copy(recv_hbm.at[pl.ds(rbase, CH)], recv_v)
            pltpu.sync_copy(cft_hbm.at[pl.ds(rbase, CH)], cf_v)

            issue_gather(0, 0)
            issue_gather(1, 1)

            def _ring(i, carry):
                process(2 * i, 0, jj)
                process(2 * i + 1, 1, jj)
                return carry
            lax.fori_loop(0, CH // 2, _ring, 0)
            wait_scatter(0)
            wait_scatter(1)
        plsc.subcore_barrier()

        for ch in range(NPT // OCH):
            rr = sid * NPT + ch * OCH
            pltpu.sync_copy(A_sh.at[pl.ds(rr, OCH)], hb0)
            pltpu.sync_copy(hb0, out_hbm.at[jj].at[pl.ds(rr, OCH)])
        plsc.subcore_barrier()


def _full(shape):
    return pl.BlockSpec(shape, lambda i: tuple(0 for _ in shape))


def kernel(sender, receiver, indices, node_attrs, node_feature_in,
           edge_length_embed, edge_sph_embed,
           W_h1p, W0, b0, W1, b1, W2, b2, W3, b3, W_mix, Wsym, W_h2, W_m2):
    f32 = jnp.float32

    Kmat = jnp.repeat(jnp.eye(S, dtype=f32), J, axis=1)
    Mmat = jnp.tile(jnp.eye(J, dtype=f32), (S, 1)) * (1.0 / AVG)
    coeff = pl.pallas_call(
        _edge_coeff_body,
        grid=(E // BE,),
        in_specs=[
            pl.BlockSpec((BE, R), lambda i: (i, 0)),
            pl.BlockSpec((BE, S), lambda i: (i, 0)),
            _full((R, 32)), _full((32,)),
            _full((32, 32)), _full((32,)),
            _full((32, 32)), _full((32,)),
            _full((32, S * J)), _full((S * J,)),
            _full((S, S * J)), _full((S * J, J)),
        ],
        out_specs=pl.BlockSpec((BE, J), lambda i: (i, 0)),
        out_shape=jax.ShapeDtypeStruct((EPAD, J), f32),
    )(edge_length_embed, edge_sph_embed, W0, b0, W1, b1, W2, b2, W3, b3,
      Kmat, Mmat)

    h1p = pl.pallas_call(
        _h1p_body,
        grid=(N // BN,),
        in_specs=[pl.BlockSpec((BN, C), lambda i: (i, 0)), _full((C, C))],
        out_specs=pl.BlockSpec((BN, C), lambda i: (i, 0)),
        out_shape=jax.ShapeDtypeStruct((N, C), f32),
    )(node_feature_in, W_h1p)

    pad = EPAD - E
    pad_i = jnp.arange(pad, dtype=jnp.int32)
    send_r = jnp.concatenate(
        [sender.astype(jnp.int32), pad_i % N]).reshape(ROWS, B)
    recv_r = jnp.concatenate(
        [receiver.astype(jnp.int32), N + pad_i % (NPAD - N)]).reshape(ROWS, B)
    cft_r = coeff.reshape(ROWS, B * J)

    mesh = plsc.VectorSubcoreMesh(core_axis_name="c", subcore_axis_name="s")
    A = pl.kernel(
        _sc_segment_body,
        out_type=jax.ShapeDtypeStruct((J, NPAD, C), f32),
        mesh=mesh,
        scratch_types=[
            pltpu.VMEM((CH, B), jnp.int32),
            pltpu.VMEM((CH, B), jnp.int32),
            pltpu.VMEM((CH, B * J), f32),
            pltpu.VMEM((B, C), f32),
            pltpu.VMEM((B, C), f32),
            pltpu.VMEM((ZR, C), f32),
            pltpu.VMEM_SHARED((NPAD, C), f32),
            pltpu.SemaphoreType.DMA, pltpu.SemaphoreType.DMA,
            pltpu.SemaphoreType.DMA, pltpu.SemaphoreType.DMA,
        ],
    )(h1p, send_r, recv_r, cft_r)

    comp2ir = jnp.array([0, 1, 1, 1])
    Wg = jnp.transpose(W_mix[comp2ir], (0, 1, 3, 2)).reshape(J, C, Z * C)
    Wsym_t = jnp.transpose(Wsym, (3, 1, 0, 2)).reshape(5 * C, Z * C)
    onehot = jax.nn.one_hot(indices, Z, dtype=f32)

    out = pl.pallas_call(
        _node_out_body,
        grid=(N // BN,),
        in_specs=[
            pl.BlockSpec((J, BN, C), lambda i: (0, i, 0)),
            pl.BlockSpec((BN, Z), lambda i: (i, 0)),
            pl.BlockSpec((BN, Z), lambda i: (i, 0)),
            pl.BlockSpec((BN, C), lambda i: (i, 0)),
            _full((J, C, Z * C)),
            _full((5 * C, Z * C)),
            _full((C, C)), _full((C, C)),
        ],
        out_specs=pl.BlockSpec((BN, C), lambda i: (i, 0)),
        out_shape=jax.ShapeDtypeStruct((N, C), f32),
    )(A, node_attrs, onehot, node_feature_in, Wg, Wsym_t, W_h2, W_m2)
    return out
